# Initial kernel scaffold; baseline (speedup 1.0000x reference)
import jax
import jax.numpy as jnp
from jax.experimental import pallas as pl


def kernel(affinity):
    raise NotImplementedError("write your pallas kernel here")


# speedup vs baseline: 2.7586x; 2.7586x over previous
import functools

import jax
import jax.numpy as jnp
from jax.experimental import pallas as pl

_K1 = 31
_NEG_INF = float("-inf")


def _kth_key(a, idx, axis, n):
    tv = jnp.max(a, axis=axis, keepdims=True)
    ti = jnp.min(jnp.where(a == tv, idx, n), axis=axis, keepdims=True)

    def body(_, carry):
        tv, ti = carry
        below = (a < tv) | ((a == tv) & (idx > ti))
        vm = jnp.where(below, a, _NEG_INF)
        tv2 = jnp.max(vm, axis=axis, keepdims=True)
        ti2 = jnp.min(jnp.where(below & (a == tv2), idx, n), axis=axis, keepdims=True)
        return tv2, ti2

    return jax.lax.fori_loop(0, _K1 - 1, body, (tv, ti))


def _row_thr_kernel(a_ref, tv_ref, ti_ref):
    a = a_ref[...]
    n = a.shape[1]
    idx = jax.lax.broadcasted_iota(jnp.int32, a.shape, 1)
    tv, ti = _kth_key(a, idx, 1, n)
    tv_ref[...] = tv
    ti_ref[...] = ti


def _col_mask_kernel(a_ref, rtv_ref, rti_ref, out_ref):
    a = a_ref[...]
    n, c = a.shape
    ridx = jax.lax.broadcasted_iota(jnp.int32, a.shape, 0)
    ctv, cti = _kth_key(a, ridx, 0, n)

    rtv = rtv_ref[...]
    rti = rti_ref[...]
    cidx = jax.lax.broadcasted_iota(jnp.int32, a.shape, 1) + pl.program_id(0) * c
    keep_r = (a > rtv) | ((a == rtv) & (cidx <= rti))
    keep_c = (a > ctv) | ((a == ctv) & (ridx <= cti))
    keep = keep_r & keep_c & (ridx != cidx)
    out_ref[...] = jnp.where(keep, a, 0.0)


def kernel(affinity):
    n = affinity.shape[0]
    blk_r = 512
    blk_c = 256

    rtv, rti = pl.pallas_call(
        _row_thr_kernel,
        grid=(n // blk_r,),
        in_specs=[pl.BlockSpec((blk_r, n), lambda i: (i, 0))],
        out_specs=[
            pl.BlockSpec((blk_r, 1), lambda i: (i, 0)),
            pl.BlockSpec((blk_r, 1), lambda i: (i, 0)),
        ],
        out_shape=[
            jax.ShapeDtypeStruct((n, 1), affinity.dtype),
            jax.ShapeDtypeStruct((n, 1), jnp.int32),
        ],
    )(affinity)

    out = pl.pallas_call(
        _col_mask_kernel,
        grid=(n // blk_c,),
        in_specs=[
            pl.BlockSpec((n, blk_c), lambda j: (0, j)),
            pl.BlockSpec((n, 1), lambda j: (0, 0)),
            pl.BlockSpec((n, 1), lambda j: (0, 0)),
        ],
        out_specs=pl.BlockSpec((n, blk_c), lambda j: (0, j)),
        out_shape=jax.ShapeDtypeStruct((n, n), affinity.dtype),
    )(affinity, rtv, rti)

    return out


# speedup vs baseline: 4.3432x; 1.5744x over previous
import jax
import jax.numpy as jnp
from jax.experimental import pallas as pl
from jax.experimental.pallas import tpu as pltpu

_K1 = 31
_NEG_INF = float("-inf")


def _kth_stat(a, idx, axis):
    tv = jnp.max(a, axis=axis, keepdims=True)
    n_ax = a.shape[axis]
    t0 = jnp.full_like(tv, _NEG_INF)
    s0 = jnp.zeros_like(tv, dtype=jnp.int32)
    prev0 = jnp.zeros_like(s0)

    def body(_, carry):
        tv, t, s, prev_cum = carry
        below = a < tv
        cum = n_ax - jnp.sum(below, axis=axis, keepdims=True, dtype=jnp.int32)
        crossed = jnp.logical_and(prev_cum < _K1, cum >= _K1)
        t = jnp.where(crossed, tv, t)
        s = jnp.where(crossed, _K1 - prev_cum, s)
        vm = jnp.where(below, a, _NEG_INF)
        tv2 = jnp.max(vm, axis=axis, keepdims=True)
        return tv2, t, s, cum

    tv, t, s, prev_cum = jax.lax.fori_loop(0, _K1, body, (tv, t0, s0, prev0))

    eq = a == t
    n = a.shape[axis]
    lo = jnp.full_like(s, -1)
    hi = jnp.full_like(s, n - 1)

    def bs_body(_, carry):
        lo, hi = carry
        mid = (lo + hi) // 2
        c = jnp.sum(
            jnp.logical_and(eq, idx <= mid), axis=axis, keepdims=True, dtype=jnp.int32
        )
        pred = c >= s
        return jnp.where(pred, lo, mid), jnp.where(pred, mid, hi)

    nbits = max(1, (n - 1).bit_length())
    lo, hi = jax.lax.fori_loop(0, nbits + 1, bs_body, (lo, hi))
    return t, hi


def _row_thr_kernel(a_ref, t_ref, ji_ref):
    a = a_ref[...]
    idx = jax.lax.broadcasted_iota(jnp.int32, a.shape, 1)
    t, ji = _kth_stat(a, idx, 1)
    t_ref[...] = t
    ji_ref[...] = ji


def _col_mask_kernel(a_ref, rt_ref, rji_ref, out_ref):
    a = a_ref[...]
    c = a.shape[1]
    ridx = jax.lax.broadcasted_iota(jnp.int32, a.shape, 0)
    ct, cji = _kth_stat(a, ridx, 0)

    rt = rt_ref[...]
    rji = rji_ref[...]
    cidx = jax.lax.broadcasted_iota(jnp.int32, a.shape, 1) + pl.program_id(0) * c
    keep_r = (a > rt) | ((a == rt) & (cidx <= rji))
    keep_c = (a > ct) | ((a == ct) & (ridx <= cji))
    keep = keep_r & keep_c & (ridx != cidx)
    out_ref[...] = jnp.where(keep, a, 0.0)


def kernel(affinity):
    n = affinity.shape[0]
    blk_r = 512
    blk_c = 256

    rt, rji = pl.pallas_call(
        _row_thr_kernel,
        grid=(n // blk_r,),
        in_specs=[pl.BlockSpec((blk_r, n), lambda i: (i, 0))],
        out_specs=[
            pl.BlockSpec((blk_r, 1), lambda i: (i, 0)),
            pl.BlockSpec((blk_r, 1), lambda i: (i, 0)),
        ],
        out_shape=[
            jax.ShapeDtypeStruct((n, 1), affinity.dtype),
            jax.ShapeDtypeStruct((n, 1), jnp.int32),
        ],
        compiler_params=pltpu.CompilerParams(dimension_semantics=("parallel",)),
    )(affinity)

    out = pl.pallas_call(
        _col_mask_kernel,
        grid=(n // blk_c,),
        in_specs=[
            pl.BlockSpec((n, blk_c), lambda j: (0, j)),
            pl.BlockSpec((n, 1), lambda j: (0, 0)),
            pl.BlockSpec((n, 1), lambda j: (0, 0)),
        ],
        out_specs=pl.BlockSpec((n, blk_c), lambda j: (0, j)),
        out_shape=jax.ShapeDtypeStruct((n, n), affinity.dtype),
        compiler_params=pltpu.CompilerParams(dimension_semantics=("parallel",)),
    )(affinity, rt, rji)

    return out


# speedup vs baseline: 5.7757x; 1.3298x over previous
import jax
import jax.numpy as jnp
from jax.experimental import pallas as pl
from jax.experimental.pallas import tpu as pltpu

_K1 = 31


def _sort_key(a):
    x = jax.lax.bitcast_convert_type(a, jnp.int32)
    return jnp.where(x < 0, x ^ jnp.int32(0x7FFFFFFF), x)


def _kth_key(k, idx, axis):
    lo = jnp.min(k, axis=axis, keepdims=True)
    hi = jnp.max(k, axis=axis, keepdims=True) + 1

    def vbody(_, carry):
        lo, hi = carry
        mid = (lo >> 1) + (hi >> 1) + (lo & hi & 1)
        c = jnp.sum(k >= mid, axis=axis, keepdims=True, dtype=jnp.int32)
        big = c >= _K1
        return jnp.where(big, mid, lo), jnp.where(big, hi, mid)

    lo, hi = jax.lax.fori_loop(0, 32, vbody, (lo, hi))
    t = lo
    s = _K1 - jnp.sum(k > t, axis=axis, keepdims=True, dtype=jnp.int32)

    eq = k == t
    n = k.shape[axis]
    jlo = jnp.full_like(s, -1)
    jhi = jnp.full_like(s, n - 1)

    def ibody(_, carry):
        jlo, jhi = carry
        mid = jlo + ((jhi - jlo) >> 1)
        c = jnp.sum(
            jnp.logical_and(eq, idx <= mid), axis=axis, keepdims=True, dtype=jnp.int32
        )
        pred = c >= s
        return jnp.where(pred, jlo, mid), jnp.where(pred, mid, jhi)

    nbits = max(1, (n - 1).bit_length())
    jlo, jhi = jax.lax.fori_loop(0, nbits, ibody, (jlo, jhi))
    return t, jhi


def _row_thr_kernel(a_ref, t_ref, ji_ref):
    k = _sort_key(a_ref[...])
    idx = jax.lax.broadcasted_iota(jnp.int32, k.shape, 1)
    t, ji = _kth_key(k, idx, 1)
    t_ref[...] = t
    ji_ref[...] = ji


def _col_mask_kernel(a_ref, rt_ref, rji_ref, out_ref):
    a = a_ref[...]
    k = _sort_key(a)
    c = a.shape[1]
    ridx = jax.lax.broadcasted_iota(jnp.int32, a.shape, 0)
    ct, cji = _kth_key(k, ridx, 0)

    rt = rt_ref[...]
    rji = rji_ref[...]
    cidx = jax.lax.broadcasted_iota(jnp.int32, a.shape, 1) + pl.program_id(0) * c
    keep_r = (k > rt) | ((k == rt) & (cidx <= rji))
    keep_c = (k > ct) | ((k == ct) & (ridx <= cji))
    keep = keep_r & keep_c & (ridx != cidx)
    out_ref[...] = jnp.where(keep, a, 0.0)


def kernel(affinity):
    n = affinity.shape[0]
    blk_r = 512
    blk_c = 256

    rt, rji = pl.pallas_call(
        _row_thr_kernel,
        grid=(n // blk_r,),
        in_specs=[pl.BlockSpec((blk_r, n), lambda i: (i, 0))],
        out_specs=[
            pl.BlockSpec((blk_r, 1), lambda i: (i, 0)),
            pl.BlockSpec((blk_r, 1), lambda i: (i, 0)),
        ],
        out_shape=[
            jax.ShapeDtypeStruct((n, 1), jnp.int32),
            jax.ShapeDtypeStruct((n, 1), jnp.int32),
        ],
        compiler_params=pltpu.CompilerParams(dimension_semantics=("parallel",)),
    )(affinity)

    out = pl.pallas_call(
        _col_mask_kernel,
        grid=(n // blk_c,),
        in_specs=[
            pl.BlockSpec((n, blk_c), lambda j: (0, j)),
            pl.BlockSpec((n, 1), lambda j: (0, 0)),
            pl.BlockSpec((n, 1), lambda j: (0, 0)),
        ],
        out_specs=pl.BlockSpec((n, blk_c), lambda j: (0, j)),
        out_shape=jax.ShapeDtypeStruct((n, n), affinity.dtype),
        compiler_params=pltpu.CompilerParams(dimension_semantics=("parallel",)),
    )(affinity, rt, rji)

    return out


# speedup vs baseline: 7.6889x; 1.3312x over previous
import jax
import jax.numpy as jnp
from jax.experimental import pallas as pl
from jax.experimental.pallas import tpu as pltpu

_K1 = 31
_IMIN = -(2**31)
_IMAX = 2**31 - 1


def _sort_key(a):
    x = jax.lax.bitcast_convert_type(a, jnp.int32)
    return jnp.where(x < 0, x ^ jnp.int32(0x7FFFFFFF), x)


def _kth_key(k, idx, axis, ji_ref):
    n = k.shape[axis]
    shape1 = tuple(1 if d == axis else s for d, s in enumerate(k.shape))
    lo = jnp.full(shape1, _IMIN, jnp.int32)
    hi = jnp.full(shape1, _IMAX, jnp.int32)
    clo = jnp.full(shape1, n, jnp.int32)
    chi = jnp.zeros(shape1, jnp.int32)

    def vbody(_, carry):
        lo, hi, clo, chi = carry
        mid = (lo >> 1) + (hi >> 1) + (lo & hi & 1)
        c = jnp.sum(k >= mid, axis=axis, keepdims=True, dtype=jnp.int32)
        big = c >= _K1
        return (
            jnp.where(big, mid, lo),
            jnp.where(big, hi, mid),
            jnp.where(big, c, clo),
            jnp.where(big, chi, c),
        )

    lo, hi, clo, chi = jax.lax.fori_loop(0, 32, vbody, (lo, hi, clo, chi))
    t = lo
    s = _K1 - chi

    ji_ref[...] = jnp.full(shape1, n - 1, jnp.int32)
    split = clo > _K1

    @pl.when(jnp.any(split))
    def _slow_ji():
        masked_idx = jnp.where(k == t, idx, n)
        jlo = jnp.full(shape1, -1, jnp.int32)
        jhi = jnp.full(shape1, n - 1, jnp.int32)

        def ibody(_, carry):
            jlo, jhi = carry
            mid = jlo + ((jhi - jlo) >> 1)
            c = jnp.sum(masked_idx <= mid, axis=axis, keepdims=True, dtype=jnp.int32)
            pred = c >= s
            return jnp.where(pred, jlo, mid), jnp.where(pred, mid, jhi)

        nbits = max(1, (n - 1).bit_length())
        jlo, jhi = jax.lax.fori_loop(0, nbits, ibody, (jlo, jhi))
        ji_ref[...] = jnp.where(split, jhi, n - 1)

    return t


def _row_thr_kernel(a_ref, t_ref, ji_ref):
    k = _sort_key(a_ref[...])
    idx = jax.lax.broadcasted_iota(jnp.int32, k.shape, 1)
    t_ref[...] = _kth_key(k, idx, 1, ji_ref)


def _col_mask_kernel(a_ref, rt_ref, rji_ref, out_ref, cji_ref):
    a = a_ref[...]
    k = _sort_key(a)
    c = a.shape[1]
    ridx = jax.lax.broadcasted_iota(jnp.int32, a.shape, 0)
    ct = _kth_key(k, ridx, 0, cji_ref)
    cji = cji_ref[...]

    rt = rt_ref[...]
    rji = rji_ref[...]
    cidx = jax.lax.broadcasted_iota(jnp.int32, a.shape, 1) + pl.program_id(0) * c
    keep_r = (k > rt) | ((k == rt) & (cidx <= rji))
    keep_c = (k > ct) | ((k == ct) & (ridx <= cji))
    keep = keep_r & keep_c & (ridx != cidx)
    out_ref[...] = jnp.where(keep, a, 0.0)


def kernel(affinity):
    n = affinity.shape[0]
    blk_r = 512
    blk_c = 256

    rt, rji = pl.pallas_call(
        _row_thr_kernel,
        grid=(n // blk_r,),
        in_specs=[pl.BlockSpec((blk_r, n), lambda i: (i, 0))],
        out_specs=[
            pl.BlockSpec((blk_r, 1), lambda i: (i, 0)),
            pl.BlockSpec((blk_r, 1), lambda i: (i, 0)),
        ],
        out_shape=[
            jax.ShapeDtypeStruct((n, 1), jnp.int32),
            jax.ShapeDtypeStruct((n, 1), jnp.int32),
        ],
    )(affinity)

    out, _ = pl.pallas_call(
        _col_mask_kernel,
        grid=(n // blk_c,),
        in_specs=[
            pl.BlockSpec((n, blk_c), lambda j: (0, j)),
            pl.BlockSpec((n, 1), lambda j: (0, 0)),
            pl.BlockSpec((n, 1), lambda j: (0, 0)),
        ],
        out_specs=[
            pl.BlockSpec((n, blk_c), lambda j: (0, j)),
            pl.BlockSpec((1, blk_c), lambda j: (0, j)),
        ],
        out_shape=[
            jax.ShapeDtypeStruct((n, n), affinity.dtype),
            jax.ShapeDtypeStruct((1, n), jnp.int32),
        ],
    )(affinity, rt, rji)

    return out


# speedup vs baseline: 8.6932x; 1.1306x over previous
import jax
import jax.numpy as jnp
from jax.experimental import pallas as pl
from jax.experimental.pallas import tpu as pltpu

_K1 = 31
_LO0 = -2139095042
_HI0 = 2139095041


def _key_to_f32(m):
    return jax.lax.bitcast_convert_type(
        jnp.where(m < 0, m ^ jnp.int32(0x7FFFFFFF), m), jnp.float32
    )


def _kth_stat(a, idx, axis, ji_ref):
    n = a.shape[axis]
    shape1 = tuple(1 if d == axis else s for d, s in enumerate(a.shape))
    lo = jnp.full(shape1, _LO0, jnp.int32)
    hi = jnp.full(shape1, _HI0, jnp.int32)
    clo = jnp.full(shape1, n, jnp.int32)
    chi = jnp.zeros(shape1, jnp.int32)

    def vbody(_, carry):
        lo, hi, clo, chi = carry
        mid = (lo >> 1) + (hi >> 1) + (lo & hi & 1)
        c = jnp.sum(a >= _key_to_f32(mid), axis=axis, keepdims=True, dtype=jnp.int32)
        big = c >= _K1
        return (
            jnp.where(big, mid, lo),
            jnp.where(big, hi, mid),
            jnp.where(big, c, clo),
            jnp.where(big, chi, c),
        )

    lo, hi, clo, chi = jax.lax.fori_loop(0, 32, vbody, (lo, hi, clo, chi))
    t = _key_to_f32(lo)
    s = _K1 - chi

    ji_ref[...] = jnp.full(shape1, n - 1, jnp.int32)
    split = clo > _K1

    @pl.when(jnp.any(split))
    def _slow_ji():
        masked_idx = jnp.where(a == t, idx, n)
        jlo = jnp.full(shape1, -1, jnp.int32)
        jhi = jnp.full(shape1, n - 1, jnp.int32)

        def ibody(_, carry):
            jlo, jhi = carry
            mid = jlo + ((jhi - jlo) >> 1)
            c = jnp.sum(masked_idx <= mid, axis=axis, keepdims=True, dtype=jnp.int32)
            pred = c >= s
            return jnp.where(pred, jlo, mid), jnp.where(pred, mid, jhi)

        nbits = max(1, (n - 1).bit_length())
        jlo, jhi = jax.lax.fori_loop(0, nbits, ibody, (jlo, jhi))
        ji_ref[...] = jnp.where(split, jhi, n - 1)

    return t


def _row_thr_kernel(a_ref, t_ref, ji_ref):
    a = a_ref[...]
    idx = jax.lax.broadcasted_iota(jnp.int32, a.shape, 1)
    t_ref[...] = _kth_stat(a, idx, 1, ji_ref)


def _col_mask_kernel(a_ref, rt_ref, rji_ref, out_ref, cji_ref):
    a = a_ref[...]
    c = a.shape[1]
    ridx = jax.lax.broadcasted_iota(jnp.int32, a.shape, 0)
    ct = _kth_stat(a, ridx, 0, cji_ref)
    cji = cji_ref[...]

    rt = rt_ref[...]
    rji = rji_ref[...]
    cidx = jax.lax.broadcasted_iota(jnp.int32, a.shape, 1) + pl.program_id(0) * c
    keep_r = (a > rt) | ((a == rt) & (cidx <= rji))
    keep_c = (a > ct) | ((a == ct) & (ridx <= cji))
    keep = keep_r & keep_c & (ridx != cidx)
    out_ref[...] = jnp.where(keep, a, 0.0)


def kernel(affinity):
    n = affinity.shape[0]
    blk_r = 512
    blk_c = 512

    rt, rji = pl.pallas_call(
        _row_thr_kernel,
        grid=(n // blk_r,),
        in_specs=[pl.BlockSpec((blk_r, n), lambda i: (i, 0))],
        out_specs=[
            pl.BlockSpec((blk_r, 1), lambda i: (i, 0)),
            pl.BlockSpec((blk_r, 1), lambda i: (i, 0)),
        ],
        out_shape=[
            jax.ShapeDtypeStruct((n, 1), affinity.dtype),
            jax.ShapeDtypeStruct((n, 1), jnp.int32),
        ],
    )(affinity)

    out, _ = pl.pallas_call(
        _col_mask_kernel,
        grid=(n // blk_c,),
        in_specs=[
            pl.BlockSpec((n, blk_c), lambda j: (0, j)),
            pl.BlockSpec((n, 1), lambda j: (0, 0)),
            pl.BlockSpec((n, 1), lambda j: (0, 0)),
        ],
        out_specs=[
            pl.BlockSpec((n, blk_c), lambda j: (0, j)),
            pl.BlockSpec((1, blk_c), lambda j: (0, j)),
        ],
        out_shape=[
            jax.ShapeDtypeStruct((n, n), affinity.dtype),
            jax.ShapeDtypeStruct((1, n), jnp.int32),
        ],
    )(affinity, rt, rji)

    return out
